# baseline (speedup 1.0000x reference)
import functools

import jax
import jax.numpy as jnp
from jax import lax
from jax.experimental import pallas as pl
from jax.experimental.pallas import tpu as pltpu
from jax.experimental.pallas import tpu_sc as plsc

N = 10000
E = 320000
D = 128
DH = D // 2

NC = 2
NS = 16
CHUNK = 128
SUP = 32
SB = 5
ROWS = SUP * SB
EPAD = NS * ROWS * CHUNK
NPAD = 10240
ROWS_T = NPAD // NS
ROWS_X = N // NS
NBUF = 2


def _sc_body(src_h, dst_h, xh_h, zs_h, zc_h, ones_h, psum_h, pcnt_h,
             src_sup, dst_sup, bufs, ones_v, x_s, acc_s, acc_c,
             gsems, ssems, csem):
    cid = lax.axis_index("c")
    sid = lax.axis_index("s")

    pltpu.sync_copy(xh_h.at[pl.ds(sid * ROWS_X, ROWS_X), pl.ds(cid * DH, DH)],
                    x_s.at[pl.ds(sid * ROWS_X, ROWS_X)])
    pltpu.sync_copy(zs_h, acc_s.at[pl.ds(sid * ROWS_T, ROWS_T)])
    pltpu.sync_copy(zc_h, acc_c.at[pl.ds(sid * ROWS_T, ROWS_T)])
    pltpu.sync_copy(ones_h, ones_v)
    plsc.subcore_barrier()

    def sb_body(sb, carry):
        pltpu.sync_copy(src_h.at[sid, pl.ds(sb * SUP, SUP)], src_sup)
        pltpu.sync_copy(dst_h.at[sid, pl.ds(sb * SUP, SUP)], dst_sup)

        for b in range(NBUF):
            pltpu.async_copy(x_s.at[src_sup.at[b]], bufs.at[b], gsems.at[b])

        def step(g, carry2):
            for b in range(NBUF):
                k = g + b
                pb = (b - 1) % NBUF
                pltpu.make_async_copy(x_s.at[src_sup.at[k]], bufs.at[b],
                                      gsems.at[b]).wait()
                pltpu.async_copy(bufs.at[b], acc_s.at[dst_sup.at[k]],
                                 ssems.at[b], add=True)
                if b % 2 == 0:
                    @pl.when(cid == 0)
                    def _():
                        pltpu.async_copy(ones_v, acc_c.at[dst_sup.at[k]],
                                         csem, add=True)
                else:
                    @pl.when(cid == 1)
                    def _():
                        pltpu.async_copy(ones_v, acc_c.at[dst_sup.at[k]],
                                         csem, add=True)

                @pl.when(k >= 1)
                def _():
                    pltpu.make_async_copy(bufs.at[pb], acc_s.at[dst_sup.at[0]],
                                          ssems.at[pb]).wait()

                @pl.when((k >= 1) & (k + 1 < SUP))
                def _():
                    pltpu.async_copy(x_s.at[src_sup.at[k + 1]], bufs.at[pb],
                                     gsems.at[pb])

            return carry2

        lax.fori_loop(0, SUP // NBUF, lambda i, c2: step(i * NBUF, c2), 0)

        pltpu.make_async_copy(bufs.at[(SUP - 1) % NBUF],
                              acc_s.at[dst_sup.at[0]],
                              ssems.at[(SUP - 1) % NBUF]).wait()

        def drain_counts(i, c2):
            pltpu.make_async_copy(ones_v, acc_c.at[dst_sup.at[0]],
                                  csem).wait()
            return c2

        lax.fori_loop(0, SUP // 2, drain_counts, 0)
        return carry

    lax.fori_loop(0, SB, sb_body, 0)
    plsc.subcore_barrier()

    pltpu.sync_copy(acc_s.at[pl.ds(sid * ROWS_T, ROWS_T)],
                    psum_h.at[cid, pl.ds(sid * ROWS_T, ROWS_T)])
    pltpu.sync_copy(acc_c.at[pl.ds(sid * ROWS_T, ROWS_T)],
                    pcnt_h.at[cid, pl.ds(sid * ROWS_T, ROWS_T)])


_sc_call = functools.partial(
    pl.kernel,
    out_type=[
        jax.ShapeDtypeStruct((NC, NPAD, DH), jnp.float32),
        jax.ShapeDtypeStruct((NC, NPAD, 16), jnp.float32),
    ],
    mesh=plsc.VectorSubcoreMesh(core_axis_name="c", subcore_axis_name="s"),
    compiler_params=pltpu.CompilerParams(use_tc_tiling_on_sc=False),
    scratch_types=[
        pltpu.VMEM((SUP, CHUNK), jnp.int32),
        pltpu.VMEM((SUP, CHUNK), jnp.int32),
        pltpu.VMEM((NBUF, CHUNK, DH), jnp.float32),
        pltpu.VMEM((CHUNK, 16), jnp.float32),
        pltpu.VMEM_SHARED((N, DH), jnp.float32),
        pltpu.VMEM_SHARED((NPAD, DH), jnp.float32),
        pltpu.VMEM_SHARED((NPAD, 16), jnp.float32),
        pltpu.SemaphoreType.DMA((NBUF,)),
        pltpu.SemaphoreType.DMA((NBUF,)),
        pltpu.SemaphoreType.DMA,
    ],
)(_sc_body)


def _tc_r_body(x_ref, wrt_ref, bl_ref, hr_ref):
    hr_ref[...] = (jnp.dot(x_ref[...], wrt_ref[...],
                           preferred_element_type=jnp.float32)
                   + bl_ref[...][None, :])


_tc_r_call = pl.pallas_call(
    _tc_r_body,
    out_shape=jax.ShapeDtypeStruct((N, D), jnp.float32),
)


def _tc_body(psum_ref, pcnt_ref, hr_ref, wlt_ref, g_ref, b_ref, out_ref):
    s = jnp.concatenate([psum_ref[0, 0:N, :], psum_ref[1, 0:N, :]], axis=1)
    c = pcnt_ref[0, 0:N, 0:1] + pcnt_ref[1, 0:N, 0:1]
    mean = s / jnp.maximum(c, 1.0)
    h = (jnp.dot(mean, wlt_ref[...], preferred_element_type=jnp.float32)
         + hr_ref[...])
    h = jnp.maximum(h, 0.0)
    mu = jnp.mean(h, axis=0, keepdims=True)
    d = h - mu
    var = jnp.mean(d * d, axis=0, keepdims=True)
    out_ref[...] = (d * lax.rsqrt(var + 1e-5) * g_ref[...][None, :]
                    + b_ref[...][None, :])


_tc_call = pl.pallas_call(
    _tc_body,
    out_shape=jax.ShapeDtypeStruct((N, D), jnp.float32),
)


@jax.jit
def kernel(x, edge_index, W_l, b_l, W_r, gamma, beta):
    src = edge_index[0]
    dst = edge_index[1]
    pad = EPAD - E
    src3 = jnp.concatenate([src, jnp.zeros((pad,), jnp.int32)]).reshape(
        NS, ROWS, CHUNK)
    dst3 = jnp.concatenate([dst, jnp.full((pad,), N, jnp.int32)]).reshape(
        NS, ROWS, CHUNK)
    zs = jnp.zeros((ROWS_T, DH), jnp.float32)
    zc = jnp.zeros((ROWS_T, 16), jnp.float32)
    ones = jnp.ones((CHUNK, 16), jnp.float32)
    psum, pcnt = _sc_call(src3, dst3, x, zs, zc, ones)
    hr = _tc_r_call(x, W_r.T, b_l)
    return _tc_call(psum, pcnt, hr, W_l.T, gamma, beta)


# speedup vs baseline: 1.0527x; 1.0527x over previous
import functools

import jax
import jax.numpy as jnp
from jax import lax
from jax.experimental import pallas as pl
from jax.experimental.pallas import tpu as pltpu
from jax.experimental.pallas import tpu_sc as plsc

N = 10000
E = 320000
D = 128
DH = D // 2

NC = 2
NS = 16
CHUNK = 128
SUP = 32
SB = 5
ROWS = SUP * SB
EPAD = NS * ROWS * CHUNK
NPAD = 10240
ROWS_T = NPAD // NS
ROWS_X = N // NS
NBUF = 2


def _sc_body(src_h, dst_h, xh_h, zs_h, zc_h, ones_h, psum_h, pcnt_h,
             src_sup, dst_sup, bufs, ones_v, x_s, acc_s, acc_c,
             gsems, ssems, csem):
    cid = lax.axis_index("c")
    sid = lax.axis_index("s")

    pltpu.sync_copy(xh_h.at[pl.ds(sid * ROWS_X, ROWS_X), pl.ds(cid * DH, DH)],
                    x_s.at[pl.ds(sid * ROWS_X, ROWS_X)])
    pltpu.sync_copy(zs_h, acc_s.at[pl.ds(sid * ROWS_T, ROWS_T)])
    pltpu.sync_copy(zc_h, acc_c.at[pl.ds(sid * ROWS_T, ROWS_T)])
    pltpu.sync_copy(ones_h, ones_v)
    plsc.subcore_barrier()

    def sb_body(sb, carry):
        pltpu.sync_copy(src_h.at[sid, pl.ds(sb * SUP, SUP)], src_sup)
        pltpu.sync_copy(dst_h.at[sid, pl.ds(sb * SUP, SUP)], dst_sup)

        for b in range(NBUF):
            pltpu.async_copy(x_s.at[src_sup.at[b]], bufs.at[b], gsems.at[b])

        def step(g, carry2):
            for b in range(NBUF):
                k = g + b
                pb = (b - 1) % NBUF
                pltpu.make_async_copy(x_s.at[src_sup.at[k]], bufs.at[b],
                                      gsems.at[b]).wait()
                pltpu.async_copy(bufs.at[b], acc_s.at[dst_sup.at[k]],
                                 ssems.at[b], add=True)
                if b % 2 == 0:
                    @pl.when(cid == 0)
                    def _():
                        pltpu.async_copy(ones_v, acc_c.at[dst_sup.at[k]],
                                         csem, add=True)
                else:
                    @pl.when(cid == 1)
                    def _():
                        pltpu.async_copy(ones_v, acc_c.at[dst_sup.at[k]],
                                         csem, add=True)

                @pl.when(k >= 1)
                def _():
                    pltpu.make_async_copy(bufs.at[pb], acc_s.at[dst_sup.at[0]],
                                          ssems.at[pb]).wait()

                @pl.when((k >= 1) & (k + 1 < SUP))
                def _():
                    pltpu.async_copy(x_s.at[src_sup.at[k + 1]], bufs.at[pb],
                                     gsems.at[pb])

            return carry2

        lax.fori_loop(0, SUP // NBUF, lambda i, c2: step(i * NBUF, c2), 0)

        pltpu.make_async_copy(bufs.at[(SUP - 1) % NBUF],
                              acc_s.at[dst_sup.at[0]],
                              ssems.at[(SUP - 1) % NBUF]).wait()

        def drain_counts(i, c2):
            pltpu.make_async_copy(ones_v, acc_c.at[dst_sup.at[0]],
                                  csem).wait()
            return c2

        lax.fori_loop(0, SUP // 2, drain_counts, 0)
        return carry

    lax.fori_loop(0, SB, sb_body, 0)
    plsc.subcore_barrier()

    pltpu.sync_copy(acc_s.at[pl.ds(sid * ROWS_T, ROWS_T)],
                    psum_h.at[pl.ds(sid * ROWS_T, ROWS_T),
                              pl.ds(cid * DH, DH)])
    pltpu.sync_copy(acc_c.at[pl.ds(sid * ROWS_T, ROWS_T)],
                    pcnt_h.at[pl.ds(sid * ROWS_T, ROWS_T),
                              pl.ds(cid * 16, 16)])


_sc_call = functools.partial(
    pl.kernel,
    out_type=[
        jax.ShapeDtypeStruct((NPAD, D), jnp.float32),
        jax.ShapeDtypeStruct((NPAD, 32), jnp.float32),
    ],
    mesh=plsc.VectorSubcoreMesh(core_axis_name="c", subcore_axis_name="s"),
    compiler_params=pltpu.CompilerParams(use_tc_tiling_on_sc=False),
    scratch_types=[
        pltpu.VMEM((SUP, CHUNK), jnp.int32),
        pltpu.VMEM((SUP, CHUNK), jnp.int32),
        pltpu.VMEM((NBUF, CHUNK, DH), jnp.float32),
        pltpu.VMEM((CHUNK, 16), jnp.float32),
        pltpu.VMEM_SHARED((N, DH), jnp.float32),
        pltpu.VMEM_SHARED((NPAD, DH), jnp.float32),
        pltpu.VMEM_SHARED((NPAD, 16), jnp.float32),
        pltpu.SemaphoreType.DMA((NBUF,)),
        pltpu.SemaphoreType.DMA((NBUF,)),
        pltpu.SemaphoreType.DMA,
    ],
)(_sc_body)


def _tc_body(psum_ref, pcnt_ref, x_ref, wlt_ref, bl_ref, wrt_ref, g_ref, b_ref,
             out_ref):
    s = psum_ref[0:N, :]
    c = pcnt_ref[0:N, 0:1] + pcnt_ref[0:N, 16:17]
    mean = s / jnp.maximum(c, 1.0)
    h = (jnp.dot(mean, wlt_ref[...], preferred_element_type=jnp.float32)
         + jnp.dot(x_ref[...], wrt_ref[...], preferred_element_type=jnp.float32)
         + bl_ref[...][None, :])
    h = jnp.maximum(h, 0.0)
    mu = jnp.mean(h, axis=0, keepdims=True)
    d = h - mu
    var = jnp.mean(d * d, axis=0, keepdims=True)
    out_ref[...] = (d * lax.rsqrt(var + 1e-5) * g_ref[...][None, :]
                    + b_ref[...][None, :])


_tc_call = pl.pallas_call(
    _tc_body,
    out_shape=jax.ShapeDtypeStruct((N, D), jnp.float32),
)


@jax.jit
def kernel(x, edge_index, W_l, b_l, W_r, gamma, beta):
    src = edge_index[0]
    dst = edge_index[1]
    pad = EPAD - E
    src3 = jnp.concatenate([src, jnp.zeros((pad,), jnp.int32)]).reshape(
        NS, ROWS, CHUNK)
    dst3 = jnp.concatenate([dst, jnp.full((pad,), N, jnp.int32)]).reshape(
        NS, ROWS, CHUNK)
    zs = jnp.zeros((ROWS_T, DH), jnp.float32)
    zc = jnp.zeros((ROWS_T, 16), jnp.float32)
    ones = jnp.ones((CHUNK, 16), jnp.float32)
    psum, pcnt = _sc_call(src3, dst3, x, zs, zc, ones)
    return _tc_call(psum, pcnt, x, W_l.T, b_l, W_r.T, gamma, beta)
